# Initial kernel scaffold; baseline (speedup 1.0000x reference)
import jax
import jax.numpy as jnp
from jax.experimental import pallas as pl


def kernel(h, edge_index, WQ, WK, WV):
    raise NotImplementedError("write your pallas kernel here")


# speedup vs baseline: 12.4043x; 12.4043x over previous
import functools

import jax
import jax.numpy as jnp
from jax import lax
from jax.experimental import pallas as pl
from jax.experimental.pallas import tpu as pltpu
from jax.experimental.pallas import tpu_sc as plsc

N = 10000
E = 320000
IN_DIM = 128
H = 8
D = 16
HD = H * D

NC = 2
NS = 16
NW = NC * NS
EPW = E // NW
B = 80
NCHUNK = EPW // B
GROUPS = B // 16
ROWS_PER_TILE = N // NS

_SCALE = 0.25


def _proj_body(h_ref, wq_ref, wk_ref, wv_ref, q_ref, k_ref, v_ref):
    hb = h_ref[...]
    q_ref[...] = jnp.dot(hb, wq_ref[...], preferred_element_type=jnp.float32)
    k_ref[...] = jnp.dot(hb, wk_ref[...], preferred_element_type=jnp.float32) * _SCALE
    v_ref[...] = jnp.dot(hb, wv_ref[...], preferred_element_type=jnp.float32)


def _proj(h, WQ, WK, WV):
    blk = 1000
    return pl.pallas_call(
        _proj_body,
        grid=(N // blk,),
        in_specs=[
            pl.BlockSpec((blk, IN_DIM), lambda i: (i, 0)),
            pl.BlockSpec((IN_DIM, HD), lambda i: (0, 0)),
            pl.BlockSpec((IN_DIM, HD), lambda i: (0, 0)),
            pl.BlockSpec((IN_DIM, HD), lambda i: (0, 0)),
        ],
        out_specs=[
            pl.BlockSpec((blk, HD), lambda i: (i, 0)),
            pl.BlockSpec((blk, HD), lambda i: (i, 0)),
            pl.BlockSpec((blk, HD), lambda i: (i, 0)),
        ],
        out_shape=[jax.ShapeDtypeStruct((N, HD), jnp.float32)] * 3,
    )(h, WQ, WK, WV)


def _edge_body(src_hbm, dst_hbm, k_hbm, q_hbm, v_hbm, zv_hbm, zz_hbm,
               outv_hbm, outz_hbm,
               acc_v, acc_z, src_c, dst_c, krows, qrows, vrows, zrows,
               sem_k, sem_q, sem_v):
    c = lax.axis_index("c")
    s = lax.axis_index("s")
    wid = c * NS + s
    iota = lax.iota(jnp.int32, 16)

    r0 = s * ROWS_PER_TILE
    pltpu.sync_copy(zv_hbm.at[pl.ds(r0, ROWS_PER_TILE), :],
                    acc_v.at[pl.ds(r0, ROWS_PER_TILE), :])
    pltpu.sync_copy(zz_hbm.at[pl.ds(r0, ROWS_PER_TILE), :],
                    acc_z.at[pl.ds(r0, ROWS_PER_TILE), :])

    plsc.subcore_barrier()

    def chunk_body(co, carry):
        pltpu.sync_copy(src_hbm.at[wid, co], src_c)
        pltpu.sync_copy(dst_hbm.at[wid, co], dst_c)
        cp_k = pltpu.async_copy(k_hbm.at[src_c], krows, sem_k)
        cp_q = pltpu.async_copy(q_hbm.at[dst_c], qrows, sem_q)
        cp_v = pltpu.async_copy(v_hbm.at[src_c], vrows, sem_v)
        cp_k.wait()
        cp_q.wait()
        cp_v.wait()

        def group_body(g, carry2):
            rows = g * 16 + iota
            for hh in range(H):
                acc16 = jnp.zeros((16,), jnp.float32)
                for dd in range(D):
                    col = jnp.full((16,), hh * D + dd, jnp.int32)
                    kvec = plsc.load_gather(krows, [rows, col])
                    qvec = plsc.load_gather(qrows, [rows, col])
                    acc16 = acc16 + kvec * qvec
                sc = jnp.exp(jnp.clip(acc16, -5.0, 5.0))
                plsc.store_scatter(zrows, [rows, jnp.full((16,), hh, jnp.int32)], sc)
                for dd in range(D):
                    col = jnp.full((16,), hh * D + dd, jnp.int32)
                    vvec = plsc.load_gather(vrows, [rows, col])
                    plsc.store_scatter(krows, [rows, col], vvec * sc)
            return carry2

        lax.fori_loop(0, GROUPS, group_body, None)

        pltpu.sync_copy(krows, acc_v.at[dst_c], add=True)
        pltpu.sync_copy(zrows, acc_z.at[dst_c], add=True)
        return carry

    lax.fori_loop(0, NCHUNK, chunk_body, None)

    plsc.subcore_barrier()

    pltpu.sync_copy(acc_v.at[pl.ds(r0, ROWS_PER_TILE), :],
                    outv_hbm.at[c, pl.ds(r0, ROWS_PER_TILE), :])
    pltpu.sync_copy(acc_z.at[pl.ds(r0, ROWS_PER_TILE), :],
                    outz_hbm.at[c, pl.ds(r0, ROWS_PER_TILE), :])


def _edge(src3, dst3, Ks, Qh, Vh, zv, zz):
    mesh = plsc.VectorSubcoreMesh(core_axis_name="c", subcore_axis_name="s")
    fn = functools.partial(
        pl.kernel,
        out_type=[
            jax.ShapeDtypeStruct((NC, N, HD), jnp.float32),
            jax.ShapeDtypeStruct((NC, N, 16), jnp.float32),
        ],
        mesh=mesh,
        compiler_params=pltpu.CompilerParams(
            use_tc_tiling_on_sc=False, needs_layout_passes=False),
        scratch_types=[
            pltpu.VMEM_SHARED((N, HD), jnp.float32),
            pltpu.VMEM_SHARED((N, 16), jnp.float32),
            pltpu.VMEM((B,), jnp.int32),
            pltpu.VMEM((B,), jnp.int32),
            pltpu.VMEM((B, HD), jnp.float32),
            pltpu.VMEM((B, HD), jnp.float32),
            pltpu.VMEM((B, HD), jnp.float32),
            pltpu.VMEM((B, 16), jnp.float32),
            pltpu.SemaphoreType.DMA,
            pltpu.SemaphoreType.DMA,
            pltpu.SemaphoreType.DMA,
        ],
    )(_edge_body)
    return fn(src3, dst3, Ks, Qh, Vh, zv, zz)


def _finish_body(v_ref, z_ref, o_ref):
    vsum = v_ref[0] + v_ref[1]
    zsum = z_ref[0] + z_ref[1]
    i0 = lax.broadcasted_iota(jnp.int32, (16, HD), 0)
    i1 = lax.broadcasted_iota(jnp.int32, (16, HD), 1) // D
    expand = (i0 == i1).astype(jnp.float32)
    zexp = jnp.dot(zsum, expand, preferred_element_type=jnp.float32)
    o_ref[...] = vsum / zexp


def _finish(outv, outz):
    blk = 1000
    return pl.pallas_call(
        _finish_body,
        grid=(N // blk,),
        in_specs=[
            pl.BlockSpec((NC, blk, HD), lambda i: (0, i, 0)),
            pl.BlockSpec((NC, blk, 16), lambda i: (0, i, 0)),
        ],
        out_specs=pl.BlockSpec((blk, HD), lambda i: (i, 0)),
        out_shape=jax.ShapeDtypeStruct((N, HD), jnp.float32),
    )(outv, outz)


def kernel(h, edge_index, WQ, WK, WV):
    Qh, Ks, Vh = _proj(h, WQ, WK, WV)
    src3 = edge_index[0].reshape(NW, NCHUNK, B)
    dst3 = edge_index[1].reshape(NW, NCHUNK, B)
    zv = jnp.zeros((N, HD), jnp.float32)
    zz = jnp.zeros((N, 16), jnp.float32)
    outv, outz = _edge(src3, dst3, Ks, Qh, Vh, zv, zz)
    out = _finish(outv, outz)
    return out.reshape(N, H, D)


# speedup vs baseline: 12.8197x; 1.0335x over previous
import functools

import jax
import jax.numpy as jnp
from jax import lax
from jax.experimental import pallas as pl
from jax.experimental.pallas import tpu as pltpu
from jax.experimental.pallas import tpu_sc as plsc

N = 10000
E = 320000
IN_DIM = 128
H = 8
D = 16
HD = H * D

NC = 2
NS = 16
NW = NC * NS

B = 48
NCHUNK = 210
NPAIR = NCHUNK // 2
EPW = NCHUNK * B
EPAD = NW * EPW
GROUPS = B // 16

NP = 10016
ROWS_PER_TILE = NP // NS

_SCALE = 0.25


def _proj_body(h_ref, wq_ref, wk_ref, wv_ref, q_ref, k_ref, v_ref):
    hb = h_ref[...]
    q_ref[...] = jnp.dot(hb, wq_ref[...], preferred_element_type=jnp.float32)
    k_ref[...] = jnp.dot(hb, wk_ref[...], preferred_element_type=jnp.float32) * _SCALE
    v_ref[...] = jnp.dot(hb, wv_ref[...], preferred_element_type=jnp.float32)


def _proj(h, WQ, WK, WV):
    blk = NP // 4
    return pl.pallas_call(
        _proj_body,
        grid=(4,),
        in_specs=[
            pl.BlockSpec((blk, IN_DIM), lambda i: (i, 0)),
            pl.BlockSpec((IN_DIM, HD), lambda i: (0, 0)),
            pl.BlockSpec((IN_DIM, HD), lambda i: (0, 0)),
            pl.BlockSpec((IN_DIM, HD), lambda i: (0, 0)),
        ],
        out_specs=[
            pl.BlockSpec((blk, HD), lambda i: (i, 0)),
            pl.BlockSpec((blk, HD), lambda i: (i, 0)),
            pl.BlockSpec((blk, HD), lambda i: (i, 0)),
        ],
        out_shape=[jax.ShapeDtypeStruct((NP, HD), jnp.float32)] * 3,
    )(h, WQ, WK, WV)


def _edge_body(src_hbm, dst_hbm, k_hbm, q_hbm, v_hbm, zv_hbm, zz_hbm,
               outv_hbm, outz_hbm,
               acc_v, acc_z,
               src0, dst0, src1, dst1,
               k0, q0, v0, k1, q1, v1, zrows,
               sem_g0, sem_g1):
    c = lax.axis_index("c")
    s = lax.axis_index("s")
    wid = c * NS + s
    iota = lax.iota(jnp.int32, 16)

    srcb = (src0, src1)
    dstb = (dst0, dst1)
    kb = (k0, k1)
    qb = (q0, q1)
    vb = (v0, v1)
    sem_g = (sem_g0, sem_g1)

    r0 = s * ROWS_PER_TILE
    pltpu.sync_copy(zv_hbm.at[pl.ds(r0, ROWS_PER_TILE), :],
                    acc_v.at[pl.ds(r0, ROWS_PER_TILE), :])
    pltpu.sync_copy(zz_hbm.at[pl.ds(r0, ROWS_PER_TILE), :],
                    acc_z.at[pl.ds(r0, ROWS_PER_TILE), :])

    plsc.subcore_barrier()

    def stage_idx(sel, co):
        pltpu.sync_copy(src_hbm.at[wid, co], srcb[sel])
        pltpu.sync_copy(dst_hbm.at[wid, co], dstb[sel])

    def issue_gathers(sel):
        pltpu.async_copy(k_hbm.at[srcb[sel]], kb[sel], sem_g[sel])
        pltpu.async_copy(q_hbm.at[dstb[sel]], qb[sel], sem_g[sel])
        pltpu.async_copy(v_hbm.at[srcb[sel]], vb[sel], sem_g[sel])

    def wait_gathers(sel):
        dummy = k_hbm.at[pl.ds(0, B), :]
        pltpu.make_async_copy(dummy, kb[sel].at[...], sem_g[sel]).wait()
        pltpu.make_async_copy(dummy, qb[sel].at[...], sem_g[sel]).wait()
        pltpu.make_async_copy(dummy, vb[sel].at[...], sem_g[sel]).wait()

    def compute(sel):
        krows, qrows, vrows = kb[sel], qb[sel], vb[sel]

        def group_body(g, carry2):
            rows = g * 16 + iota
            for hh in range(H):
                acc16 = jnp.zeros((16,), jnp.float32)
                for dd in range(D):
                    col = jnp.full((16,), hh * D + dd, jnp.int32)
                    kvec = plsc.load_gather(krows, [rows, col])
                    qvec = plsc.load_gather(qrows, [rows, col])
                    acc16 = acc16 + kvec * qvec
                sc = jnp.exp(jnp.clip(acc16, -5.0, 5.0))
                plsc.store_scatter(zrows, [rows, jnp.full((16,), hh, jnp.int32)], sc)
                for dd in range(D):
                    col = jnp.full((16,), hh * D + dd, jnp.int32)
                    vvec = plsc.load_gather(vrows, [rows, col])
                    plsc.store_scatter(krows, [rows, col], vvec * sc)
            return carry2

        lax.fori_loop(0, GROUPS, group_body, None)

        pltpu.sync_copy(krows, acc_v.at[dstb[sel]], add=True)
        pltpu.sync_copy(zrows, acc_z.at[dstb[sel]], add=True)

    stage_idx(0, 0)
    issue_gathers(0)

    def pair_body(p, carry):
        for b in range(2):
            cur, nxt = b, 1 - b
            co = 2 * p + b
            cnext = jnp.minimum(co + 1, NCHUNK - 1)
            stage_idx(nxt, cnext)
            issue_gathers(nxt)
            wait_gathers(cur)
            compute(cur)
        return carry

    lax.fori_loop(0, NPAIR, pair_body, None)

    wait_gathers(0)

    plsc.subcore_barrier()

    pltpu.sync_copy(acc_v.at[pl.ds(r0, ROWS_PER_TILE), :],
                    outv_hbm.at[c, pl.ds(r0, ROWS_PER_TILE), :])
    pltpu.sync_copy(acc_z.at[pl.ds(r0, ROWS_PER_TILE), :],
                    outz_hbm.at[c, pl.ds(r0, ROWS_PER_TILE), :])


def _edge(src3, dst3, Ks, Qh, Vh, zv, zz):
    mesh = plsc.VectorSubcoreMesh(core_axis_name="c", subcore_axis_name="s")
    fn = functools.partial(
        pl.kernel,
        out_type=[
            jax.ShapeDtypeStruct((NC, NP, HD), jnp.float32),
            jax.ShapeDtypeStruct((NC, NP, 16), jnp.float32),
        ],
        mesh=mesh,
        compiler_params=pltpu.CompilerParams(
            use_tc_tiling_on_sc=False, needs_layout_passes=False),
        scratch_types=[
            pltpu.VMEM_SHARED((NP, HD), jnp.float32),
            pltpu.VMEM_SHARED((NP, 16), jnp.float32),
            pltpu.VMEM((B,), jnp.int32),
            pltpu.VMEM((B,), jnp.int32),
            pltpu.VMEM((B,), jnp.int32),
            pltpu.VMEM((B,), jnp.int32),
            pltpu.VMEM((B, HD), jnp.float32),
            pltpu.VMEM((B, HD), jnp.float32),
            pltpu.VMEM((B, HD), jnp.float32),
            pltpu.VMEM((B, HD), jnp.float32),
            pltpu.VMEM((B, HD), jnp.float32),
            pltpu.VMEM((B, HD), jnp.float32),
            pltpu.VMEM((B, 16), jnp.float32),
            pltpu.SemaphoreType.DMA,
            pltpu.SemaphoreType.DMA,
        ],
    )(_edge_body)
    return fn(src3, dst3, Ks, Qh, Vh, zv, zz)


def _finish_body(v_ref, z_ref, o_ref):
    vsum = v_ref[0] + v_ref[1]
    zsum = z_ref[0] + z_ref[1]
    i0 = lax.broadcasted_iota(jnp.int32, (16, HD), 0)
    i1 = lax.broadcasted_iota(jnp.int32, (16, HD), 1) // D
    expand = (i0 == i1).astype(jnp.float32)
    zexp = jnp.dot(zsum, expand, preferred_element_type=jnp.float32)
    o_ref[...] = vsum / zexp


def _finish(outv, outz):
    blk = 1000
    return pl.pallas_call(
        _finish_body,
        grid=(N // blk,),
        in_specs=[
            pl.BlockSpec((NC, blk, HD), lambda i: (0, i, 0)),
            pl.BlockSpec((NC, blk, 16), lambda i: (0, i, 0)),
        ],
        out_specs=pl.BlockSpec((blk, HD), lambda i: (i, 0)),
        out_shape=jax.ShapeDtypeStruct((N, HD), jnp.float32),
    )(outv, outz)


def kernel(h, edge_index, WQ, WK, WV):
    hp = jnp.pad(h, ((0, NP - N), (0, 0)))
    Qh, Ks, Vh = _proj(hp, WQ, WK, WV)
    pad = jnp.full((EPAD - E,), N, jnp.int32)
    src3 = jnp.concatenate([edge_index[0].astype(jnp.int32), pad]).reshape(
        NW, NCHUNK, B)
    dst3 = jnp.concatenate([edge_index[1].astype(jnp.int32), pad]).reshape(
        NW, NCHUNK, B)
    zv = jnp.zeros((NP, HD), jnp.float32)
    zz = jnp.zeros((NP, 16), jnp.float32)
    outv, outz = _edge(src3, dst3, Ks, Qh, Vh, zv, zz)
    out = _finish(outv[:, :N], outz[:, :N])
    return out.reshape(N, H, D)


# speedup vs baseline: 12.9806x; 1.0125x over previous
import functools

import jax
import jax.numpy as jnp
from jax import lax
from jax.experimental import pallas as pl
from jax.experimental.pallas import tpu as pltpu
from jax.experimental.pallas import tpu_sc as plsc

N = 10000
E = 320000
IN_DIM = 128
H = 8
D = 16
HD = H * D

NC = 2
NS = 16
NW = NC * NS

B = 48
NCHUNK = 212
NSEXT = (NCHUNK - 2) // 6
EPW = NCHUNK * B
EPAD = NW * EPW
GROUPS = B // 16

NP = 10016
ROWS_PER_TILE = NP // NS

_SCALE = 0.25


def _proj_body(h_ref, wq_ref, wk_ref, wv_ref, q_ref, k_ref, v_ref):
    hb = h_ref[...]
    q_ref[...] = jnp.dot(hb, wq_ref[...], preferred_element_type=jnp.float32)
    k_ref[...] = jnp.dot(hb, wk_ref[...], preferred_element_type=jnp.float32) * _SCALE
    v_ref[...] = jnp.dot(hb, wv_ref[...], preferred_element_type=jnp.float32)


def _proj(h, WQ, WK, WV):
    blk = NP // 4
    return pl.pallas_call(
        _proj_body,
        grid=(4,),
        in_specs=[
            pl.BlockSpec((blk, IN_DIM), lambda i: (i, 0)),
            pl.BlockSpec((IN_DIM, HD), lambda i: (0, 0)),
            pl.BlockSpec((IN_DIM, HD), lambda i: (0, 0)),
            pl.BlockSpec((IN_DIM, HD), lambda i: (0, 0)),
        ],
        out_specs=[
            pl.BlockSpec((blk, HD), lambda i: (i, 0)),
            pl.BlockSpec((blk, HD), lambda i: (i, 0)),
            pl.BlockSpec((blk, HD), lambda i: (i, 0)),
        ],
        out_shape=[jax.ShapeDtypeStruct((NP, HD), jnp.float32)] * 3,
    )(h, WQ, WK, WV)


def _edge_body(src_hbm, dst_hbm, k_hbm, q_hbm, v_hbm, zv_hbm, zz_hbm,
               outv_hbm, outz_hbm,
               acc_v, acc_z,
               src0, dst0, src1, dst1, src2, dst2,
               k0, k1, k2, q0, q1, v0, v1, z0, z1,
               sem_g0, sem_g1, sem_s0, sem_s1):
    c = lax.axis_index("c")
    s = lax.axis_index("s")
    wid = c * NS + s
    iota = lax.iota(jnp.int32, 16)

    srcb = (src0, src1, src2)
    dstb = (dst0, dst1, dst2)
    kb = (k0, k1, k2)
    qb = (q0, q1)
    vb = (v0, v1)
    zb = (z0, z1)
    sem_g = (sem_g0, sem_g1)
    sem_s = (sem_s0, sem_s1)

    r0 = s * ROWS_PER_TILE
    pltpu.sync_copy(zv_hbm.at[pl.ds(r0, ROWS_PER_TILE), :],
                    acc_v.at[pl.ds(r0, ROWS_PER_TILE), :])
    pltpu.sync_copy(zz_hbm.at[pl.ds(r0, ROWS_PER_TILE), :],
                    acc_z.at[pl.ds(r0, ROWS_PER_TILE), :])

    plsc.subcore_barrier()

    def stage_idx(ip, co):
        pltpu.sync_copy(src_hbm.at[wid, co], srcb[ip])
        pltpu.sync_copy(dst_hbm.at[wid, co], dstb[ip])

    def issue_gathers(kp, gp):
        pltpu.async_copy(k_hbm.at[srcb[kp]], kb[kp], sem_g[gp])
        pltpu.async_copy(q_hbm.at[dstb[kp]], qb[gp], sem_g[gp])
        pltpu.async_copy(v_hbm.at[srcb[kp]], vb[gp], sem_g[gp])

    def wait_gathers(gp):
        dummy = k_hbm.at[pl.ds(0, B), :]
        pltpu.make_async_copy(dummy, k0.at[...], sem_g[gp]).wait()
        pltpu.make_async_copy(dummy, q0.at[...], sem_g[gp]).wait()
        pltpu.make_async_copy(dummy, v0.at[...], sem_g[gp]).wait()

    def issue_scatter(kp, gp):
        pltpu.async_copy(kb[kp], acc_v.at[dstb[kp]], sem_s[gp], add=True)
        pltpu.async_copy(zb[gp], acc_z.at[dstb[kp]], sem_s[gp], add=True)

    def wait_scatter(gp):
        pltpu.make_async_copy(k_hbm.at[pl.ds(0, B), :], k0.at[...],
                              sem_s[gp]).wait()
        pltpu.make_async_copy(zz_hbm.at[pl.ds(0, B), :], z0.at[...],
                              sem_s[gp]).wait()

    def compute(kp, gp):
        krows, qrows, vrows, zrows = kb[kp], qb[gp], vb[gp], zb[gp]

        def group_body(g, carry2):
            rows = g * 16 + iota
            for hh in range(H):
                acc16 = jnp.zeros((16,), jnp.float32)
                for dd in range(D):
                    col = jnp.full((16,), hh * D + dd, jnp.int32)
                    kvec = plsc.load_gather(krows, [rows, col])
                    qvec = plsc.load_gather(qrows, [rows, col])
                    acc16 = acc16 + kvec * qvec
                sc = jnp.exp(jnp.clip(acc16, -5.0, 5.0))
                plsc.store_scatter(zrows, [rows, jnp.full((16,), hh, jnp.int32)], sc)
                for dd in range(D):
                    col = jnp.full((16,), hh * D + dd, jnp.int32)
                    vvec = plsc.load_gather(vrows, [rows, col])
                    plsc.store_scatter(krows, [rows, col], vvec * sc)
            return carry2

        lax.fori_loop(0, GROUPS, group_body, None)

    stage_idx(0, 0)
    issue_gathers(0, 0)

    stage_idx(1, 1)
    issue_gathers(1, 1)
    wait_gathers(0)
    compute(0, 0)
    issue_scatter(0, 0)

    stage_idx(2, 2)
    issue_gathers(2, 0)
    wait_gathers(1)
    compute(1, 1)
    issue_scatter(1, 1)

    def six_body(i, carry):
        base = 2 + 6 * i
        for j in range(6):
            kp, gp = (2 + j) % 3, j % 2
            kp1, gp1 = j % 3, (j + 1) % 2
            co = base + j
            wait_scatter(gp)
            cnext = jnp.minimum(co + 1, NCHUNK - 1)
            stage_idx(kp1, cnext)
            issue_gathers(kp1, gp1)
            wait_gathers(gp)
            compute(kp, gp)
            issue_scatter(kp, gp)
        return carry

    lax.fori_loop(0, NSEXT, six_body, None)

    wait_gathers(0)
    wait_scatter(0)
    wait_scatter(1)

    plsc.subcore_barrier()

    pltpu.sync_copy(acc_v.at[pl.ds(r0, ROWS_PER_TILE), :],
                    outv_hbm.at[c, pl.ds(r0, ROWS_PER_TILE), :])
    pltpu.sync_copy(acc_z.at[pl.ds(r0, ROWS_PER_TILE), :],
                    outz_hbm.at[c, pl.ds(r0, ROWS_PER_TILE), :])


def _edge(src3, dst3, Ks, Qh, Vh, zv, zz):
    mesh = plsc.VectorSubcoreMesh(core_axis_name="c", subcore_axis_name="s")
    fn = functools.partial(
        pl.kernel,
        out_type=[
            jax.ShapeDtypeStruct((NC, NP, HD), jnp.float32),
            jax.ShapeDtypeStruct((NC, NP, 8), jnp.float32),
        ],
        mesh=mesh,
        compiler_params=pltpu.CompilerParams(
            use_tc_tiling_on_sc=False, needs_layout_passes=False),
        scratch_types=[
            pltpu.VMEM_SHARED((NP, HD), jnp.float32),
            pltpu.VMEM_SHARED((NP, 8), jnp.float32),
            pltpu.VMEM((B,), jnp.int32),
            pltpu.VMEM((B,), jnp.int32),
            pltpu.VMEM((B,), jnp.int32),
            pltpu.VMEM((B,), jnp.int32),
            pltpu.VMEM((B,), jnp.int32),
            pltpu.VMEM((B,), jnp.int32),
            pltpu.VMEM((B, HD), jnp.float32),
            pltpu.VMEM((B, HD), jnp.float32),
            pltpu.VMEM((B, HD), jnp.float32),
            pltpu.VMEM((B, HD), jnp.float32),
            pltpu.VMEM((B, HD), jnp.float32),
            pltpu.VMEM((B, HD), jnp.float32),
            pltpu.VMEM((B, HD), jnp.float32),
            pltpu.VMEM((B, 8), jnp.float32),
            pltpu.VMEM((B, 8), jnp.float32),
            pltpu.SemaphoreType.DMA,
            pltpu.SemaphoreType.DMA,
            pltpu.SemaphoreType.DMA,
            pltpu.SemaphoreType.DMA,
        ],
    )(_edge_body)
    return fn(src3, dst3, Ks, Qh, Vh, zv, zz)


def _finish_body(v_ref, z_ref, o_ref):
    vsum = v_ref[0] + v_ref[1]
    zsum = z_ref[0] + z_ref[1]
    i0 = lax.broadcasted_iota(jnp.int32, (8, HD), 0)
    i1 = lax.broadcasted_iota(jnp.int32, (8, HD), 1) // D
    expand = (i0 == i1).astype(jnp.float32)
    zexp = jnp.dot(zsum, expand, preferred_element_type=jnp.float32)
    o_ref[...] = vsum / zexp


def _finish(outv, outz):
    blk = 1000
    return pl.pallas_call(
        _finish_body,
        grid=(N // blk,),
        in_specs=[
            pl.BlockSpec((NC, blk, HD), lambda i: (0, i, 0)),
            pl.BlockSpec((NC, blk, 8), lambda i: (0, i, 0)),
        ],
        out_specs=pl.BlockSpec((blk, HD), lambda i: (i, 0)),
        out_shape=jax.ShapeDtypeStruct((N, HD), jnp.float32),
    )(outv, outz)


def kernel(h, edge_index, WQ, WK, WV):
    hp = jnp.pad(h, ((0, NP - N), (0, 0)))
    Qh, Ks, Vh = _proj(hp, WQ, WK, WV)
    pad = jnp.full((EPAD - E,), N, jnp.int32)
    src3 = jnp.concatenate([edge_index[0].astype(jnp.int32), pad]).reshape(
        NW, NCHUNK, B)
    dst3 = jnp.concatenate([edge_index[1].astype(jnp.int32), pad]).reshape(
        NW, NCHUNK, B)
    zv = jnp.zeros((NP, HD), jnp.float32)
    zz = jnp.zeros((NP, 8), jnp.float32)
    outv, outz = _edge(src3, dst3, Ks, Qh, Vh, zv, zz)
    out = _finish(outv[:, :N], outz[:, :N])
    return out.reshape(N, H, D)


# speedup vs baseline: 13.2042x; 1.0172x over previous
import functools

import jax
import jax.numpy as jnp
from jax import lax
from jax.experimental import pallas as pl
from jax.experimental.pallas import tpu as pltpu
from jax.experimental.pallas import tpu_sc as plsc

N = 10000
E = 320000
IN_DIM = 128
H = 8
D = 16
HD = H * D

NC = 2
NS = 16
NW = NC * NS

B = 48
NCHUNK = 212
NSEXT = (NCHUNK - 2) // 6
EPW = NCHUNK * B
EPAD = NW * EPW
GROUPS = B // 16

NP = 10016
ROWS_PER_TILE = NP // NS

_SCALE = 0.25


def _proj_body(h_ref, wq_ref, wk_ref, wv_ref, q_ref, k_ref, v_ref):
    hb = h_ref[...]
    q_ref[...] = jnp.dot(hb, wq_ref[...], preferred_element_type=jnp.float32)
    k_ref[...] = jnp.dot(hb, wk_ref[...], preferred_element_type=jnp.float32) * _SCALE
    v_ref[...] = jnp.dot(hb, wv_ref[...], preferred_element_type=jnp.float32)


def _proj(h, WQ, WK, WV):
    blk = NP // 4
    return pl.pallas_call(
        _proj_body,
        grid=(4,),
        in_specs=[
            pl.BlockSpec((blk, IN_DIM), lambda i: (i, 0)),
            pl.BlockSpec((IN_DIM, HD), lambda i: (0, 0)),
            pl.BlockSpec((IN_DIM, HD), lambda i: (0, 0)),
            pl.BlockSpec((IN_DIM, HD), lambda i: (0, 0)),
        ],
        out_specs=[
            pl.BlockSpec((blk, HD), lambda i: (i, 0)),
            pl.BlockSpec((blk, HD), lambda i: (i, 0)),
            pl.BlockSpec((blk, HD), lambda i: (i, 0)),
        ],
        out_shape=[jax.ShapeDtypeStruct((NP, HD), jnp.float32)] * 3,
    )(h, WQ, WK, WV)


def _edge_body(src_hbm, dst_hbm, k_hbm, q_hbm, v_hbm, zv_hbm, zz_hbm,
               outv_hbm, outz_hbm,
               acc_v, acc_z,
               src0, dst0, src1, dst1, src2, dst2,
               k0, k1, k2, q0, q1, v0, v1, z0, z1,
               sem_g0, sem_g1, sem_s0, sem_s1):
    c = lax.axis_index("c")
    s = lax.axis_index("s")
    wid = c * NS + s
    iota = lax.iota(jnp.int32, 16)

    srcb = (src0, src1, src2)
    dstb = (dst0, dst1, dst2)
    kb = (k0, k1, k2)
    qb = (q0, q1)
    vb = (v0, v1)
    zb = (z0, z1)
    sem_g = (sem_g0, sem_g1)
    sem_s = (sem_s0, sem_s1)

    r0 = s * ROWS_PER_TILE
    pltpu.sync_copy(zv_hbm.at[pl.ds(r0, ROWS_PER_TILE), :],
                    acc_v.at[pl.ds(r0, ROWS_PER_TILE), :])
    pltpu.sync_copy(zz_hbm.at[pl.ds(r0, ROWS_PER_TILE), :],
                    acc_z.at[pl.ds(r0, ROWS_PER_TILE), :])

    plsc.subcore_barrier()

    def stage_idx(ip, co):
        pltpu.sync_copy(src_hbm.at[wid, co], srcb[ip])
        pltpu.sync_copy(dst_hbm.at[wid, co], dstb[ip])

    def issue_gathers(kp, gp):
        pltpu.async_copy(k_hbm.at[srcb[kp]], kb[kp], sem_g[gp])
        pltpu.async_copy(q_hbm.at[dstb[kp]], qb[gp], sem_g[gp])
        pltpu.async_copy(v_hbm.at[srcb[kp]], vb[gp], sem_g[gp])

    def wait_gathers(gp):
        dummy = k_hbm.at[pl.ds(0, B), :]
        pltpu.make_async_copy(dummy, k0.at[...], sem_g[gp]).wait()
        pltpu.make_async_copy(dummy, q0.at[...], sem_g[gp]).wait()
        pltpu.make_async_copy(dummy, v0.at[...], sem_g[gp]).wait()

    def issue_scatter(kp, gp):
        pltpu.async_copy(kb[kp], acc_v.at[dstb[kp]], sem_s[gp], add=True)
        pltpu.async_copy(zb[gp], acc_z.at[dstb[kp]], sem_s[gp], add=True)

    def wait_scatter(gp):
        pltpu.make_async_copy(k_hbm.at[pl.ds(0, B), :], k0.at[...],
                              sem_s[gp]).wait()
        pltpu.make_async_copy(zz_hbm.at[pl.ds(0, B), :], z0.at[...],
                              sem_s[gp]).wait()

    def compute(kp, gp):
        krows, qrows, vrows, zrows = kb[kp], qb[gp], vb[gp], zb[gp]

        def group_body(g, carry2):
            rows = g * 16 + iota
            for h0 in range(0, H, 2):
                h1 = h0 + 1
                acc0 = jnp.zeros((16,), jnp.float32)
                acc1 = jnp.zeros((16,), jnp.float32)
                for dd in range(D):
                    col0 = jnp.full((16,), h0 * D + dd, jnp.int32)
                    col1 = jnp.full((16,), h1 * D + dd, jnp.int32)
                    acc0 = acc0 + (plsc.load_gather(krows, [rows, col0]) *
                                   plsc.load_gather(qrows, [rows, col0]))
                    acc1 = acc1 + (plsc.load_gather(krows, [rows, col1]) *
                                   plsc.load_gather(qrows, [rows, col1]))
                sc0 = jnp.exp(jnp.clip(acc0, -5.0, 5.0))
                sc1 = jnp.exp(jnp.clip(acc1, -5.0, 5.0))
                plsc.store_scatter(zrows, [rows, jnp.full((16,), h0, jnp.int32)], sc0)
                plsc.store_scatter(zrows, [rows, jnp.full((16,), h1, jnp.int32)], sc1)
                for dd in range(D):
                    col0 = jnp.full((16,), h0 * D + dd, jnp.int32)
                    col1 = jnp.full((16,), h1 * D + dd, jnp.int32)
                    plsc.store_scatter(krows, [rows, col0],
                                       plsc.load_gather(vrows, [rows, col0]) * sc0)
                    plsc.store_scatter(krows, [rows, col1],
                                       plsc.load_gather(vrows, [rows, col1]) * sc1)
            return carry2

        lax.fori_loop(0, GROUPS, group_body, None)

    stage_idx(0, 0)
    issue_gathers(0, 0)

    stage_idx(1, 1)
    issue_gathers(1, 1)
    wait_gathers(0)
    compute(0, 0)
    issue_scatter(0, 0)

    stage_idx(2, 2)
    issue_gathers(2, 0)
    wait_gathers(1)
    compute(1, 1)
    issue_scatter(1, 1)

    def six_body(i, carry):
        base = 2 + 6 * i
        for j in range(6):
            kp, gp = (2 + j) % 3, j % 2
            kp1, gp1 = j % 3, (j + 1) % 2
            co = base + j
            wait_scatter(gp)
            cnext = jnp.minimum(co + 1, NCHUNK - 1)
            stage_idx(kp1, cnext)
            issue_gathers(kp1, gp1)
            wait_gathers(gp)
            compute(kp, gp)
            issue_scatter(kp, gp)
        return carry

    lax.fori_loop(0, NSEXT, six_body, None)

    wait_gathers(0)
    wait_scatter(0)
    wait_scatter(1)

    plsc.subcore_barrier()

    pltpu.sync_copy(acc_v.at[pl.ds(r0, ROWS_PER_TILE), :],
                    outv_hbm.at[c, pl.ds(r0, ROWS_PER_TILE), :])
    pltpu.sync_copy(acc_z.at[pl.ds(r0, ROWS_PER_TILE), :],
                    outz_hbm.at[c, pl.ds(r0, ROWS_PER_TILE), :])


def _edge(src3, dst3, Ks, Qh, Vh, zv, zz):
    mesh = plsc.VectorSubcoreMesh(core_axis_name="c", subcore_axis_name="s")
    fn = functools.partial(
        pl.kernel,
        out_type=[
            jax.ShapeDtypeStruct((NC, NP, HD), jnp.float32),
            jax.ShapeDtypeStruct((NC, NP, 8), jnp.float32),
        ],
        mesh=mesh,
        compiler_params=pltpu.CompilerParams(
            use_tc_tiling_on_sc=False, needs_layout_passes=False),
        scratch_types=[
            pltpu.VMEM_SHARED((NP, HD), jnp.float32),
            pltpu.VMEM_SHARED((NP, 8), jnp.float32),
            pltpu.VMEM((B,), jnp.int32),
            pltpu.VMEM((B,), jnp.int32),
            pltpu.VMEM((B,), jnp.int32),
            pltpu.VMEM((B,), jnp.int32),
            pltpu.VMEM((B,), jnp.int32),
            pltpu.VMEM((B,), jnp.int32),
            pltpu.VMEM((B, HD), jnp.float32),
            pltpu.VMEM((B, HD), jnp.float32),
            pltpu.VMEM((B, HD), jnp.float32),
            pltpu.VMEM((B, HD), jnp.float32),
            pltpu.VMEM((B, HD), jnp.float32),
            pltpu.VMEM((B, HD), jnp.float32),
            pltpu.VMEM((B, HD), jnp.float32),
            pltpu.VMEM((B, 8), jnp.float32),
            pltpu.VMEM((B, 8), jnp.float32),
            pltpu.SemaphoreType.DMA,
            pltpu.SemaphoreType.DMA,
            pltpu.SemaphoreType.DMA,
            pltpu.SemaphoreType.DMA,
        ],
    )(_edge_body)
    return fn(src3, dst3, Ks, Qh, Vh, zv, zz)


def _finish_body(v_ref, z_ref, o_ref):
    vsum = v_ref[0] + v_ref[1]
    zsum = z_ref[0] + z_ref[1]
    i0 = lax.broadcasted_iota(jnp.int32, (8, HD), 0)
    i1 = lax.broadcasted_iota(jnp.int32, (8, HD), 1) // D
    expand = (i0 == i1).astype(jnp.float32)
    zexp = jnp.dot(zsum, expand, preferred_element_type=jnp.float32)
    o_ref[...] = vsum / zexp


def _finish(outv, outz):
    blk = 1000
    return pl.pallas_call(
        _finish_body,
        grid=(N // blk,),
        in_specs=[
            pl.BlockSpec((NC, blk, HD), lambda i: (0, i, 0)),
            pl.BlockSpec((NC, blk, 8), lambda i: (0, i, 0)),
        ],
        out_specs=pl.BlockSpec((blk, HD), lambda i: (i, 0)),
        out_shape=jax.ShapeDtypeStruct((N, HD), jnp.float32),
    )(outv, outz)


def kernel(h, edge_index, WQ, WK, WV):
    hp = jnp.pad(h, ((0, NP - N), (0, 0)))
    Qh, Ks, Vh = _proj(hp, WQ, WK, WV)
    pad = jnp.full((EPAD - E,), N, jnp.int32)
    src3 = jnp.concatenate([edge_index[0].astype(jnp.int32), pad]).reshape(
        NW, NCHUNK, B)
    dst3 = jnp.concatenate([edge_index[1].astype(jnp.int32), pad]).reshape(
        NW, NCHUNK, B)
    zv = jnp.zeros((NP, HD), jnp.float32)
    zz = jnp.zeros((NP, 8), jnp.float32)
    outv, outz = _edge(src3, dst3, Ks, Qh, Vh, zv, zz)
    out = _finish(outv[:, :N], outz[:, :N])
    return out.reshape(N, H, D)
